# baseline (speedup 1.0000x reference)
import functools

import jax
import jax.numpy as jnp
from jax import lax
from jax.experimental import pallas as pl
from jax.experimental.pallas import tpu as pltpu
from jax.experimental.pallas import tpu_sc as plsc

_VOCAB = 100000
_MAXLEN = 2048
_EMB = 768
_HEMB = _EMB // 2
_COND = 128
_B = 4
_S = 2048
_EPS = 1e-12

_NTOK = _B * _S
_NC = 2
_NS = 16
_NW = _NC * _NS
_C = 16
_LANES = 16


def _sc_gather_sum(ids_flat, pos_flat, ids_table, pos_table, ntok):
    per_w = ntok // _NW
    nchunk = per_w // _C
    mesh = plsc.VectorSubcoreMesh(core_axis_name="c", subcore_axis_name="s")

    @functools.partial(
        pl.kernel,
        mesh=mesh,
        out_type=jax.ShapeDtypeStruct((ntok, _HEMB), jnp.int32),
        scratch_types=[
            pltpu.VMEM((per_w,), jnp.int32),
            pltpu.VMEM((per_w,), jnp.int32),
            pltpu.VMEM((2, _C, _EMB), jnp.float32),
            pltpu.VMEM((2, _C, _EMB), jnp.float32),
            pltpu.VMEM((2, _C, _HEMB), jnp.int32),
            pltpu.SemaphoreType.DMA,
            pltpu.SemaphoreType.DMA,
            pltpu.SemaphoreType.DMA,
            pltpu.SemaphoreType.DMA,
            pltpu.SemaphoreType.DMA,
            pltpu.SemaphoreType.DMA,
        ],
    )
    def sc_kernel(ids_hbm, pos_hbm, idtab_hbm, postab_hbm, out_hbm,
                  idx_i, idx_p, rows_i, rows_p, rows_w,
                  sem_gi0, sem_gi1, sem_gp0, sem_gp1, sem_w0, sem_w1):
        sem_gi = (sem_gi0, sem_gi1)
        sem_gp = (sem_gp0, sem_gp1)
        sem_w = (sem_w0, sem_w1)
        wid = lax.axis_index("s") * _NC + lax.axis_index("c")
        base = wid * per_w

        pltpu.sync_copy(ids_hbm.at[pl.ds(base, per_w)], idx_i)
        pltpu.sync_copy(pos_hbm.at[pl.ds(base, per_w)], idx_p)

        def fire_gathers(c):
            s = c % 2
            gi = pltpu.async_copy(
                idtab_hbm.at[idx_i.at[pl.ds(c * _C, _C)]], rows_i.at[s],
                sem_gi[s])
            gp = pltpu.async_copy(
                postab_hbm.at[idx_p.at[pl.ds(c * _C, _C)]], rows_p.at[s],
                sem_gp[s])
            return gi, gp

        def bf16_pack(a, b):
            au = lax.bitcast_convert_type(a, jnp.int32)
            bu = lax.bitcast_convert_type(b, jnp.int32)
            rnd = jnp.full((_LANES,), 0x8000, jnp.int32)
            msk = jnp.full((_LANES,), -65536, jnp.int32)
            lo = lax.shift_right_logical(au + rnd, jnp.full((_LANES,), 16, jnp.int32))
            hi = lax.bitwise_and(bu + rnd, msk)
            return lax.bitwise_or(lo, hi)

        pend_g = [None, None]
        pend_w = [None, None]
        pend_g[0] = fire_gathers(0)

        for c in range(nchunk):
            s = c % 2
            if c + 1 < nchunk:
                pend_g[1 - s] = fire_gathers(c + 1)
            gi, gp = pend_g[s]
            gi.wait()
            gp.wait()
            if pend_w[s] is not None:
                pend_w[s].wait()

            def row_body(r, c2):
                for v in range(_HEMB // _LANES):
                    sl = pl.ds(v * _LANES, _LANES)
                    sh = pl.ds(_HEMB + v * _LANES, _LANES)
                    a = rows_i[s, r, sl] + rows_p[s, r, sl]
                    b = rows_i[s, r, sh] + rows_p[s, r, sh]
                    rows_w[s, r, sl] = bf16_pack(a, b)
                return c2

            lax.fori_loop(0, _C, row_body, 0)
            pend_w[s] = pltpu.async_copy(
                rows_w.at[s], out_hbm.at[pl.ds(base + c * _C, _C)], sem_w[s])
        pend_w[0].wait()
        pend_w[1].wait()

    return sc_kernel(ids_flat, pos_flat, ids_table, pos_table)


_TBLK = 512


def _tc_body(g_ref, cond_ref, w_ref, b_ref, scale_ref, bias_ref, o_ref):
    w = g_ref[...]
    first = lax.bitcast_convert_type(w << 16, jnp.float32)
    second = lax.bitcast_convert_type(w & jnp.int32(-65536), jnp.float32)
    g = jnp.concatenate([first, second], axis=-1)
    proj = jnp.dot(cond_ref[...], w_ref[...], preferred_element_type=jnp.float32)
    x = g + jnp.tanh(proj + b_ref[...])
    mu = jnp.mean(x, axis=-1, keepdims=True)
    xc = x - mu
    var = jnp.mean(xc * xc, axis=-1, keepdims=True)
    o_ref[...] = xc * lax.rsqrt(var + _EPS) * scale_ref[...] + bias_ref[...]


def _tc_fuse(gathered, cond2d, cond_W, cond_b, ln_scale, ln_bias):
    grid = (_NTOK // _TBLK,)
    return pl.pallas_call(
        _tc_body,
        grid=grid,
        in_specs=[
            pl.BlockSpec((_TBLK, _HEMB), lambda i: (i, 0)),
            pl.BlockSpec((_TBLK, _COND), lambda i: (i, 0)),
            pl.BlockSpec((_COND, _EMB), lambda i: (0, 0)),
            pl.BlockSpec((1, _EMB), lambda i: (0, 0)),
            pl.BlockSpec((1, _EMB), lambda i: (0, 0)),
            pl.BlockSpec((1, _EMB), lambda i: (0, 0)),
        ],
        out_specs=pl.BlockSpec((_TBLK, _EMB), lambda i: (i, 0)),
        out_shape=jax.ShapeDtypeStruct((_NTOK, _EMB), jnp.float32),
    )(gathered, cond2d, cond_W,
      cond_b.reshape(1, _EMB), ln_scale.reshape(1, _EMB),
      ln_bias.reshape(1, _EMB))


def kernel(ids, conditions, pos_ids, ids_table, pos_table, cond_W, cond_b,
           ln_scale, ln_bias):
    ids_flat = ids.reshape(_NTOK)
    pos_flat = pos_ids.reshape(_NTOK)
    gathered = _sc_gather_sum(ids_flat, pos_flat, ids_table, pos_table, _NTOK)
    y = _tc_fuse(gathered, conditions.reshape(_NTOK, _COND), cond_W, cond_b,
                 ln_scale, ln_bias)
    return y.reshape(_B, _S, _EMB)


# speedup vs baseline: 1.3833x; 1.3833x over previous
import functools

import jax
import jax.numpy as jnp
from jax import lax
from jax.experimental import pallas as pl
from jax.experimental.pallas import tpu as pltpu
from jax.experimental.pallas import tpu_sc as plsc

_VOCAB = 100000
_MAXLEN = 2048
_EMB = 768
_HEMB = _EMB // 2
_COND = 128
_B = 4
_S = 2048
_EPS = 1e-12

_NTOK = _B * _S
_NC = 2
_NS = 16
_NW = _NC * _NS
_C = 32
_LANES = 16


def _sc_gather_sum(ids_flat, pos_flat, ids_table, pos_table, ntok):
    per_w = ntok // _NW
    nchunk = per_w // _C
    npair = nchunk // 2
    mesh = plsc.VectorSubcoreMesh(core_axis_name="c", subcore_axis_name="s")

    @functools.partial(
        pl.kernel,
        mesh=mesh,
        out_type=jax.ShapeDtypeStruct((ntok, _HEMB), jnp.int32),
        scratch_types=[
            pltpu.VMEM((per_w,), jnp.int32),
            pltpu.VMEM((per_w,), jnp.int32),
            pltpu.VMEM((2, _C, _EMB), jnp.float32),
            pltpu.VMEM((2, _C, _EMB), jnp.float32),
            pltpu.VMEM((2, _C, _HEMB), jnp.int32),
            pltpu.SemaphoreType.DMA,
            pltpu.SemaphoreType.DMA,
            pltpu.SemaphoreType.DMA,
            pltpu.SemaphoreType.DMA,
            pltpu.SemaphoreType.DMA,
            pltpu.SemaphoreType.DMA,
        ],
    )
    def sc_kernel(ids_hbm, pos_hbm, idtab_hbm, postab_hbm, out_hbm,
                  idx_i, idx_p, rows_i, rows_p, rows_w,
                  sem_gi0, sem_gi1, sem_gp0, sem_gp1, sem_w0, sem_w1):
        sem_gi = (sem_gi0, sem_gi1)
        sem_gp = (sem_gp0, sem_gp1)
        sem_w = (sem_w0, sem_w1)
        wid = lax.axis_index("s") * _NC + lax.axis_index("c")
        base = wid * per_w

        pltpu.sync_copy(ids_hbm.at[pl.ds(base, per_w)], idx_i)
        pltpu.sync_copy(pos_hbm.at[pl.ds(base, per_w)], idx_p)

        def fire_gathers(c, s):
            pltpu.async_copy(
                idtab_hbm.at[idx_i.at[pl.ds(c * _C, _C)]], rows_i.at[s],
                sem_gi[s])
            pltpu.async_copy(
                postab_hbm.at[idx_p.at[pl.ds(c * _C, _C)]], rows_p.at[s],
                sem_gp[s])

        def wait_gathers(s):
            pltpu.make_async_copy(
                idtab_hbm.at[pl.ds(0, _C)], rows_i.at[s], sem_gi[s]).wait()
            pltpu.make_async_copy(
                postab_hbm.at[pl.ds(0, _C)], rows_p.at[s], sem_gp[s]).wait()

        def fire_write(c, s):
            pltpu.async_copy(
                rows_w.at[s], out_hbm.at[pl.ds(base + c * _C, _C)], sem_w[s])

        def wait_write(s):
            pltpu.make_async_copy(
                rows_w.at[s], out_hbm.at[pl.ds(0, _C)], sem_w[s]).wait()

        msk = jnp.full((_LANES,), -65536, jnp.int32)
        sh16 = jnp.full((_LANES,), 16, jnp.int32)

        def compute(s):
            @plsc.parallel_loop(0, _C, 1, unroll=2)
            def row_body(r):
                for v in range(_HEMB // _LANES):
                    sl = pl.ds(v * _LANES, _LANES)
                    sh = pl.ds(_HEMB + v * _LANES, _LANES)
                    a = rows_i[s, r, sl] + rows_p[s, r, sl]
                    b = rows_i[s, r, sh] + rows_p[s, r, sh]
                    au = lax.bitcast_convert_type(a, jnp.int32)
                    bu = lax.bitcast_convert_type(b, jnp.int32)
                    lo = lax.shift_right_logical(au, sh16)
                    hi = lax.bitwise_and(bu, msk)
                    rows_w[s, r, sl] = lax.bitwise_or(lo, hi)

        fire_gathers(0, 0)

        def pair_body(p, carry):
            c = 2 * p
            fire_gathers(c + 1, 1)
            wait_gathers(0)

            @pl.when(p > 0)
            def _():
                wait_write(0)

            compute(0)
            fire_write(c, 0)

            @pl.when(p + 1 < npair)
            def _():
                fire_gathers(c + 2, 0)

            wait_gathers(1)

            @pl.when(p > 0)
            def _():
                wait_write(1)

            compute(1)
            fire_write(c + 1, 1)
            return carry

        lax.fori_loop(0, npair, pair_body, 0)
        wait_write(0)
        wait_write(1)

    return sc_kernel(ids_flat, pos_flat, ids_table, pos_table)


_TBLK = 512


def _tc_body(g_ref, cond_ref, w_ref, b_ref, scale_ref, bias_ref, o_ref):
    w = g_ref[...]
    first = lax.bitcast_convert_type(w << 16, jnp.float32)
    second = lax.bitcast_convert_type(w & jnp.int32(-65536), jnp.float32)
    g = jnp.concatenate([first, second], axis=-1)
    proj = jnp.dot(cond_ref[...], w_ref[...], preferred_element_type=jnp.float32)
    x = g + jnp.tanh(proj + b_ref[...])
    mu = jnp.mean(x, axis=-1, keepdims=True)
    xc = x - mu
    var = jnp.mean(xc * xc, axis=-1, keepdims=True)
    o_ref[...] = xc * lax.rsqrt(var + _EPS) * scale_ref[...] + bias_ref[...]


def _tc_fuse(gathered, cond2d, cond_W, cond_b, ln_scale, ln_bias):
    grid = (_NTOK // _TBLK,)
    return pl.pallas_call(
        _tc_body,
        grid=grid,
        in_specs=[
            pl.BlockSpec((_TBLK, _HEMB), lambda i: (i, 0)),
            pl.BlockSpec((_TBLK, _COND), lambda i: (i, 0)),
            pl.BlockSpec((_COND, _EMB), lambda i: (0, 0)),
            pl.BlockSpec((1, _EMB), lambda i: (0, 0)),
            pl.BlockSpec((1, _EMB), lambda i: (0, 0)),
            pl.BlockSpec((1, _EMB), lambda i: (0, 0)),
        ],
        out_specs=pl.BlockSpec((_TBLK, _EMB), lambda i: (i, 0)),
        out_shape=jax.ShapeDtypeStruct((_NTOK, _EMB), jnp.float32),
    )(gathered, cond2d, cond_W,
      cond_b.reshape(1, _EMB), ln_scale.reshape(1, _EMB),
      ln_bias.reshape(1, _EMB))


def kernel(ids, conditions, pos_ids, ids_table, pos_table, cond_W, cond_b,
           ln_scale, ln_bias):
    ids_flat = ids.reshape(_NTOK)
    pos_flat = pos_ids.reshape(_NTOK)
    gathered = _sc_gather_sum(ids_flat, pos_flat, ids_table, pos_table, _NTOK)
    y = _tc_fuse(gathered, conditions.reshape(_NTOK, _COND), cond_W, cond_b,
                 ln_scale, ln_bias)
    return y.reshape(_B, _S, _EMB)
